# baseline (speedup 1.0000x reference)
import functools

import jax
import jax.numpy as jnp
from jax import lax
from jax.experimental import pallas as pl
from jax.experimental.pallas import tpu as pltpu
from jax.experimental.pallas import tpu_sc as plsc

N_NODES = 10000
N_EDGES = 320000
IN_F = 128
H_F = 32
NC = 2
NS = 16
NW = NC * NS
EPW = 10240
E_PAD = EPW * NW
K = 128
NG = EPW // K
NPAD = 10240
N_DUMMY = NPAD - N_NODES
RPT = NPAD // NS
BR = 1000


@functools.cache
def _sc_mesh():
    return plsc.VectorSubcoreMesh(core_axis_name="c", subcore_axis_name="s")


@functools.cache
def _sc_deg():
    @functools.partial(
        pl.kernel,
        out_type=jax.ShapeDtypeStruct((NC, NPAD, 8), jnp.float32),
        mesh=_sc_mesh(),
        scratch_types=[
            pltpu.VMEM_SHARED((NPAD, 8), jnp.float32),
            pltpu.VMEM((NG, K), jnp.int32),
            pltpu.VMEM((K, 8), jnp.float32),
        ],
        compiler_params=pltpu.CompilerParams(use_tc_tiling_on_sc=False),
    )
    def deg_kernel(dst_hbm, ones_hbm, zeros_hbm, out_hbm, acc, dst_v, ones_v):
        c = lax.axis_index("c")
        s = lax.axis_index("s")
        w = c * NS + s
        pltpu.sync_copy(zeros_hbm.at[pl.ds(s * RPT, RPT)],
                        acc.at[pl.ds(s * RPT, RPT)])
        pltpu.sync_copy(dst_hbm.at[w], dst_v)
        pltpu.sync_copy(ones_hbm, ones_v)
        plsc.subcore_barrier()

        def body(j, carry):
            pltpu.sync_copy(ones_v, acc.at[dst_v.at[j]], add=True)
            return carry

        lax.fori_loop(0, NG, body, 0)
        plsc.subcore_barrier()
        pltpu.sync_copy(acc.at[pl.ds(s * RPT, RPT)],
                        out_hbm.at[c, pl.ds(s * RPT, RPT)])

    return deg_kernel


@functools.cache
def _sc_lap():
    @functools.partial(
        pl.kernel,
        out_type=jax.ShapeDtypeStruct((NC, NPAD, H_F), jnp.float32),
        mesh=_sc_mesh(),
        scratch_types=[
            pltpu.VMEM_SHARED((NPAD, H_F), jnp.float32),
            pltpu.VMEM((NG, K), jnp.int32),
            pltpu.VMEM((NG, K), jnp.int32),
            pltpu.VMEM((K, H_F), jnp.float32),
            pltpu.SemaphoreType.DMA,
        ],
        compiler_params=pltpu.CompilerParams(use_tc_tiling_on_sc=False),
    )
    def lap_kernel(g_hbm, src_hbm, dst_hbm, zeros_hbm, out_hbm,
                   acc, src_v, dst_v, rows_v, sem):
        c = lax.axis_index("c")
        s = lax.axis_index("s")
        w = c * NS + s
        pltpu.sync_copy(zeros_hbm.at[pl.ds(s * RPT, RPT)],
                        acc.at[pl.ds(s * RPT, RPT)])
        pltpu.sync_copy(src_hbm.at[w], src_v)
        pltpu.sync_copy(dst_hbm.at[w], dst_v)
        plsc.subcore_barrier()

        def body(j, carry):
            pltpu.async_copy(g_hbm.at[src_v.at[j]], rows_v, sem).wait()
            pltpu.sync_copy(rows_v, acc.at[dst_v.at[j]], add=True)
            return carry

        lax.fori_loop(0, NG, body, 0)
        plsc.subcore_barrier()
        pltpu.sync_copy(acc.at[pl.ds(s * RPT, RPT)],
                        out_hbm.at[c, pl.ds(s * RPT, RPT)])

    return lap_kernel


def _dinv_from(deg_ref):
    deg = deg_ref[0] + deg_ref[1]
    return lax.rsqrt(jnp.maximum(deg[:, 0:1], 1.0))


def _bdot(a, b):
    return jnp.dot(a.astype(jnp.bfloat16), b.astype(jnp.bfloat16),
                   preferred_element_type=jnp.float32)


def _tc_pre_body(x_ref, w1_ref, b1_ref, w2_ref, b2_ref, deg_ref, h_ref, g_ref):
    h1 = jnp.maximum(_bdot(x_ref[...], w1_ref[...]) + b1_ref[...], 0.0)
    h2 = jnp.maximum(_bdot(h1, w2_ref[...]) + b2_ref[...], 0.0)
    h_ref[...] = h2
    g_ref[...] = h2 * _dinv_from(deg_ref)


def _tc_pre(x, W1, b1, W2, b2, deg_parts):
    grid = N_NODES // BR
    return pl.pallas_call(
        _tc_pre_body,
        grid=(grid,),
        in_specs=[
            pl.BlockSpec((BR, IN_F), lambda i: (i, 0)),
            pl.BlockSpec((IN_F, H_F), lambda i: (0, 0)),
            pl.BlockSpec((1, H_F), lambda i: (0, 0)),
            pl.BlockSpec((H_F, H_F), lambda i: (0, 0)),
            pl.BlockSpec((1, H_F), lambda i: (0, 0)),
            pl.BlockSpec((NC, BR, 8), lambda i: (0, i, 0)),
        ],
        out_specs=[
            pl.BlockSpec((BR, H_F), lambda i: (i, 0)),
            pl.BlockSpec((BR, H_F), lambda i: (i, 0)),
        ],
        out_shape=[
            jax.ShapeDtypeStruct((N_NODES, H_F), jnp.float32),
            jax.ShapeDtypeStruct((N_NODES, H_F), jnp.float32),
        ],
    )(x, W1, b1, W2, b2, deg_parts)


def _tc_mid_body(h_ref, agg_ref, deg_ref, f_ref, g2_ref):
    dinv = _dinv_from(deg_ref)
    f = h_ref[...] - (agg_ref[0] + agg_ref[1]) * dinv
    f_ref[...] = f
    g2_ref[...] = f * dinv


def _tc_mid(h, agg_parts, deg_parts):
    grid = N_NODES // BR
    return pl.pallas_call(
        _tc_mid_body,
        grid=(grid,),
        in_specs=[
            pl.BlockSpec((BR, H_F), lambda i: (i, 0)),
            pl.BlockSpec((NC, BR, H_F), lambda i: (0, i, 0)),
            pl.BlockSpec((NC, BR, 8), lambda i: (0, i, 0)),
        ],
        out_specs=[
            pl.BlockSpec((BR, H_F), lambda i: (i, 0)),
            pl.BlockSpec((BR, H_F), lambda i: (i, 0)),
        ],
        out_shape=[
            jax.ShapeDtypeStruct((N_NODES, H_F), jnp.float32),
            jax.ShapeDtypeStruct((N_NODES, H_F), jnp.float32),
        ],
    )(h, agg_parts, deg_parts)


def _tc_post_body(h_ref, f1_ref, agg_ref, deg_ref, wm1_ref, bm1_ref,
                  wm2_ref, bm2_ref, out_ref):
    dinv = _dinv_from(deg_ref)
    h = h_ref[...]
    f1 = f1_ref[...]
    f2 = f1 - (agg_ref[0] + agg_ref[1]) * dinv
    out0 = 3.0 * h + (-3.0) * f1 + 0.75 * f2
    out1 = 3.0 * f1 + (-1.5) * f2
    out2 = 0.75 * f2
    hf = jnp.concatenate([out0, out1, out2], axis=1)
    hm = jnp.maximum(_bdot(hf, wm1_ref[...]) + bm1_ref[...], 0.0)
    out_ref[...] = _bdot(hm, wm2_ref[...]) + bm2_ref[...]


def _tc_post(h, f1, agg_parts, deg_parts, Wm1, bm1, Wm2, bm2):
    grid = N_NODES // BR
    return pl.pallas_call(
        _tc_post_body,
        grid=(grid,),
        in_specs=[
            pl.BlockSpec((BR, H_F), lambda i: (i, 0)),
            pl.BlockSpec((BR, H_F), lambda i: (i, 0)),
            pl.BlockSpec((NC, BR, H_F), lambda i: (0, i, 0)),
            pl.BlockSpec((NC, BR, 8), lambda i: (0, i, 0)),
            pl.BlockSpec((3 * H_F, H_F), lambda i: (0, 0)),
            pl.BlockSpec((1, H_F), lambda i: (0, 0)),
            pl.BlockSpec((H_F, 2), lambda i: (0, 0)),
            pl.BlockSpec((1, 2), lambda i: (0, 0)),
        ],
        out_specs=pl.BlockSpec((BR, 2), lambda i: (i, 0)),
        out_shape=jax.ShapeDtypeStruct((N_NODES, 2), jnp.float32),
    )(h, f1, agg_parts, deg_parts, Wm1, bm1, Wm2, bm2)


def kernel(x, edge_index, W1, b1, W2, b2, Wm1, bm1, Wm2, bm2):
    pad_n = E_PAD - N_EDGES
    src = jnp.concatenate(
        [edge_index[0].astype(jnp.int32), jnp.zeros((pad_n,), jnp.int32)]
    ).reshape(NW, NG, K)
    dst = jnp.concatenate(
        [edge_index[1].astype(jnp.int32),
         N_NODES + (jnp.arange(pad_n, dtype=jnp.int32) % N_DUMMY)]
    ).reshape(NW, NG, K)
    zeros32 = jnp.zeros((NPAD, H_F), jnp.float32)
    zeros8 = jnp.zeros((NPAD, 8), jnp.float32)
    ones8 = jnp.ones((K, 8), jnp.float32)

    deg_parts = _sc_deg()(dst, ones8, zeros8)
    h, g = _tc_pre(x, W1, b1.reshape(1, H_F), W2, b2.reshape(1, H_F),
                   deg_parts)
    agg1 = _sc_lap()(g, src, dst, zeros32)
    f1, g2 = _tc_mid(h, agg1, deg_parts)
    agg2 = _sc_lap()(g2, src, dst, zeros32)
    return _tc_post(h, f1, agg2, deg_parts, Wm1, bm1.reshape(1, H_F),
                    Wm2, bm2.reshape(1, 2))


# speedup vs baseline: 1.8465x; 1.8465x over previous
import functools

import jax
import jax.numpy as jnp
from jax import lax
from jax.experimental import pallas as pl
from jax.experimental.pallas import tpu as pltpu
from jax.experimental.pallas import tpu_sc as plsc

N_NODES = 10000
N_EDGES = 320000
IN_F = 128
H_F = 32
NC = 2
NS = 16
NW = NC * NS
EPW = N_EDGES // NW
K = 80
CHUNKS = EPW // K
NPAD = 10240
RPT = NPAD // NS
GRT = N_NODES // NS
BR = 1000


@functools.cache
def _sc_mesh():
    return plsc.VectorSubcoreMesh(core_axis_name="c", subcore_axis_name="s")


@functools.cache
def _sc_deg():
    @functools.partial(
        pl.kernel,
        out_type=jax.ShapeDtypeStruct((NC, NPAD, 8), jnp.float32),
        mesh=_sc_mesh(),
        scratch_types=[
            pltpu.VMEM_SHARED((NPAD, 8), jnp.float32),
            pltpu.VMEM((CHUNKS, K), jnp.int32),
            pltpu.VMEM((K, 8), jnp.float32),
        ],
        compiler_params=pltpu.CompilerParams(use_tc_tiling_on_sc=False),
    )
    def deg_kernel(dst_hbm, ones_hbm, zeros_hbm, out_hbm, acc, dst_v, ones_v):
        c = lax.axis_index("c")
        s = lax.axis_index("s")
        w = c * NS + s
        pltpu.sync_copy(zeros_hbm.at[pl.ds(s * RPT, RPT)],
                        acc.at[pl.ds(s * RPT, RPT)])
        pltpu.sync_copy(dst_hbm.at[w], dst_v)
        pltpu.sync_copy(ones_hbm, ones_v)
        plsc.subcore_barrier()

        def body(j, carry):
            pltpu.sync_copy(ones_v, acc.at[dst_v.at[j]], add=True)
            return carry

        lax.fori_loop(0, CHUNKS, body, 0)
        plsc.subcore_barrier()
        pltpu.sync_copy(acc.at[pl.ds(s * RPT, RPT)],
                        out_hbm.at[c, pl.ds(s * RPT, RPT)])

    return deg_kernel


@functools.cache
def _sc_lap():
    @functools.partial(
        pl.kernel,
        out_type=jax.ShapeDtypeStruct((NC, NPAD, H_F), jnp.float32),
        mesh=_sc_mesh(),
        scratch_types=[
            pltpu.VMEM_SHARED((NPAD, H_F), jnp.float32),
            pltpu.VMEM_SHARED((N_NODES, H_F), jnp.float32),
            pltpu.VMEM((CHUNKS, K), jnp.int32),
            pltpu.VMEM((CHUNKS, K), jnp.int32),
            pltpu.VMEM((K, H_F), jnp.float32),
            pltpu.SemaphoreType.DMA,
        ],
        compiler_params=pltpu.CompilerParams(use_tc_tiling_on_sc=False),
    )
    def lap_kernel(g_hbm, src_hbm, dst_hbm, zeros_hbm, out_hbm,
                   acc, g_sp, src_v, dst_v, rows_v, sem):
        c = lax.axis_index("c")
        s = lax.axis_index("s")
        w = c * NS + s
        pltpu.sync_copy(zeros_hbm.at[pl.ds(s * RPT, RPT)],
                        acc.at[pl.ds(s * RPT, RPT)])
        pltpu.sync_copy(g_hbm.at[pl.ds(s * GRT, GRT)],
                        g_sp.at[pl.ds(s * GRT, GRT)])
        pltpu.sync_copy(src_hbm.at[w], src_v)
        pltpu.sync_copy(dst_hbm.at[w], dst_v)
        plsc.subcore_barrier()

        def body(j, carry):
            pltpu.async_copy(g_sp.at[src_v.at[j]], rows_v, sem).wait()
            pltpu.sync_copy(rows_v, acc.at[dst_v.at[j]], add=True)
            return carry

        lax.fori_loop(0, CHUNKS, body, 0)
        plsc.subcore_barrier()
        pltpu.sync_copy(acc.at[pl.ds(s * RPT, RPT)],
                        out_hbm.at[c, pl.ds(s * RPT, RPT)])

    return lap_kernel


def _dinv_from(deg_ref):
    deg = deg_ref[0] + deg_ref[1]
    return lax.rsqrt(jnp.maximum(deg[:, 0:1], 1.0))


def _bdot(a, b):
    return jnp.dot(a.astype(jnp.bfloat16), b.astype(jnp.bfloat16),
                   preferred_element_type=jnp.float32)


def _tc_pre_body(x_ref, w1_ref, b1_ref, w2_ref, b2_ref, deg_ref, h_ref, g_ref):
    h1 = jnp.maximum(_bdot(x_ref[...], w1_ref[...]) + b1_ref[...], 0.0)
    h2 = jnp.maximum(_bdot(h1, w2_ref[...]) + b2_ref[...], 0.0)
    h_ref[...] = h2
    g_ref[...] = h2 * _dinv_from(deg_ref)


def _tc_pre(x, W1, b1, W2, b2, deg_parts):
    grid = N_NODES // BR
    return pl.pallas_call(
        _tc_pre_body,
        grid=(grid,),
        in_specs=[
            pl.BlockSpec((BR, IN_F), lambda i: (i, 0)),
            pl.BlockSpec((IN_F, H_F), lambda i: (0, 0)),
            pl.BlockSpec((1, H_F), lambda i: (0, 0)),
            pl.BlockSpec((H_F, H_F), lambda i: (0, 0)),
            pl.BlockSpec((1, H_F), lambda i: (0, 0)),
            pl.BlockSpec((NC, BR, 8), lambda i: (0, i, 0)),
        ],
        out_specs=[
            pl.BlockSpec((BR, H_F), lambda i: (i, 0)),
            pl.BlockSpec((BR, H_F), lambda i: (i, 0)),
        ],
        out_shape=[
            jax.ShapeDtypeStruct((N_NODES, H_F), jnp.float32),
            jax.ShapeDtypeStruct((N_NODES, H_F), jnp.float32),
        ],
    )(x, W1, b1, W2, b2, deg_parts)


def _tc_mid_body(h_ref, agg_ref, deg_ref, f_ref, g2_ref):
    dinv = _dinv_from(deg_ref)
    f = h_ref[...] - (agg_ref[0] + agg_ref[1]) * dinv
    f_ref[...] = f
    g2_ref[...] = f * dinv


def _tc_mid(h, agg_parts, deg_parts):
    grid = N_NODES // BR
    return pl.pallas_call(
        _tc_mid_body,
        grid=(grid,),
        in_specs=[
            pl.BlockSpec((BR, H_F), lambda i: (i, 0)),
            pl.BlockSpec((NC, BR, H_F), lambda i: (0, i, 0)),
            pl.BlockSpec((NC, BR, 8), lambda i: (0, i, 0)),
        ],
        out_specs=[
            pl.BlockSpec((BR, H_F), lambda i: (i, 0)),
            pl.BlockSpec((BR, H_F), lambda i: (i, 0)),
        ],
        out_shape=[
            jax.ShapeDtypeStruct((N_NODES, H_F), jnp.float32),
            jax.ShapeDtypeStruct((N_NODES, H_F), jnp.float32),
        ],
    )(h, agg_parts, deg_parts)


def _tc_post_body(h_ref, f1_ref, agg_ref, deg_ref, wm1_ref, bm1_ref,
                  wm2_ref, bm2_ref, out_ref):
    dinv = _dinv_from(deg_ref)
    h = h_ref[...]
    f1 = f1_ref[...]
    f2 = f1 - (agg_ref[0] + agg_ref[1]) * dinv
    out0 = 3.0 * h + (-3.0) * f1 + 0.75 * f2
    out1 = 3.0 * f1 + (-1.5) * f2
    out2 = 0.75 * f2
    hf = jnp.concatenate([out0, out1, out2], axis=1)
    hm = jnp.maximum(_bdot(hf, wm1_ref[...]) + bm1_ref[...], 0.0)
    out_ref[...] = _bdot(hm, wm2_ref[...]) + bm2_ref[...]


def _tc_post(h, f1, agg_parts, deg_parts, Wm1, bm1, Wm2, bm2):
    grid = N_NODES // BR
    return pl.pallas_call(
        _tc_post_body,
        grid=(grid,),
        in_specs=[
            pl.BlockSpec((BR, H_F), lambda i: (i, 0)),
            pl.BlockSpec((BR, H_F), lambda i: (i, 0)),
            pl.BlockSpec((NC, BR, H_F), lambda i: (0, i, 0)),
            pl.BlockSpec((NC, BR, 8), lambda i: (0, i, 0)),
            pl.BlockSpec((3 * H_F, H_F), lambda i: (0, 0)),
            pl.BlockSpec((1, H_F), lambda i: (0, 0)),
            pl.BlockSpec((H_F, 2), lambda i: (0, 0)),
            pl.BlockSpec((1, 2), lambda i: (0, 0)),
        ],
        out_specs=pl.BlockSpec((BR, 2), lambda i: (i, 0)),
        out_shape=jax.ShapeDtypeStruct((N_NODES, 2), jnp.float32),
    )(h, f1, agg_parts, deg_parts, Wm1, bm1, Wm2, bm2)


def kernel(x, edge_index, W1, b1, W2, b2, Wm1, bm1, Wm2, bm2):
    src = edge_index[0].astype(jnp.int32).reshape(NW, CHUNKS, K)
    dst = edge_index[1].astype(jnp.int32).reshape(NW, CHUNKS, K)
    zeros32 = jnp.zeros((NPAD, H_F), jnp.float32)
    zeros8 = jnp.zeros((NPAD, 8), jnp.float32)
    ones8 = jnp.ones((K, 8), jnp.float32)

    deg_parts = _sc_deg()(dst, ones8, zeros8)
    h, g = _tc_pre(x, W1, b1.reshape(1, H_F), W2, b2.reshape(1, H_F),
                   deg_parts)
    agg1 = _sc_lap()(g, src, dst, zeros32)
    f1, g2 = _tc_mid(h, agg1, deg_parts)
    agg2 = _sc_lap()(g2, src, dst, zeros32)
    return _tc_post(h, f1, agg2, deg_parts, Wm1, bm1.reshape(1, H_F),
                    Wm2, bm2.reshape(1, 2))


# speedup vs baseline: 2.1235x; 1.1501x over previous
import functools

import jax
import jax.numpy as jnp
from jax import lax
from jax.experimental import pallas as pl
from jax.experimental.pallas import tpu as pltpu
from jax.experimental.pallas import tpu_sc as plsc

N_NODES = 10000
N_EDGES = 320000
IN_F = 128
H_F = 32
NC = 2
NS = 16
NW = NC * NS
EPW = N_EDGES // NW
K = 80
CHUNKS = EPW // K
NPAD = 10240
RPT = NPAD // NS
GRT = N_NODES // NS
BR = 1000


@functools.cache
def _sc_mesh():
    return plsc.VectorSubcoreMesh(core_axis_name="c", subcore_axis_name="s")


@functools.cache
def _sc_deg():
    @functools.partial(
        pl.kernel,
        out_type=jax.ShapeDtypeStruct((NC, NPAD, 8), jnp.float32),
        mesh=_sc_mesh(),
        scratch_types=[
            pltpu.VMEM_SHARED((NPAD, 8), jnp.float32),
            pltpu.VMEM((CHUNKS, K), jnp.int32),
            pltpu.VMEM((K, 8), jnp.float32),
        ],
        compiler_params=pltpu.CompilerParams(use_tc_tiling_on_sc=False),
    )
    def deg_kernel(dst_hbm, ones_hbm, zeros_hbm, out_hbm, acc, dst_v, ones_v):
        c = lax.axis_index("c")
        s = lax.axis_index("s")
        w = c * NS + s
        pltpu.sync_copy(zeros_hbm.at[pl.ds(s * RPT, RPT)],
                        acc.at[pl.ds(s * RPT, RPT)])
        pltpu.sync_copy(dst_hbm.at[w], dst_v)
        pltpu.sync_copy(ones_hbm, ones_v)
        plsc.subcore_barrier()

        def body(j, carry):
            pltpu.sync_copy(ones_v, acc.at[dst_v.at[j]], add=True)
            return carry

        lax.fori_loop(0, CHUNKS, body, 0)
        plsc.subcore_barrier()
        pltpu.sync_copy(acc.at[pl.ds(s * RPT, RPT)],
                        out_hbm.at[c, pl.ds(s * RPT, RPT)])

    return deg_kernel


@functools.cache
def _sc_lap():
    @functools.partial(
        pl.kernel,
        out_type=jax.ShapeDtypeStruct((NC, NPAD, H_F), jnp.float32),
        mesh=_sc_mesh(),
        scratch_types=[
            pltpu.VMEM_SHARED((NPAD, H_F), jnp.float32),
            pltpu.VMEM_SHARED((N_NODES, H_F), jnp.float32),
            pltpu.VMEM((CHUNKS, K), jnp.int32),
            pltpu.VMEM((CHUNKS, K), jnp.int32),
            pltpu.VMEM((K, H_F), jnp.float32),
            pltpu.VMEM((K, H_F), jnp.float32),
            pltpu.SemaphoreType.DMA,
            pltpu.SemaphoreType.DMA,
        ],
        compiler_params=pltpu.CompilerParams(use_tc_tiling_on_sc=False),
    )
    def lap_kernel(g_hbm, src_hbm, dst_hbm, zeros_hbm, out_hbm,
                   acc, g_sp, src_v, dst_v, rows_a, rows_b, sga, sgb):
        c = lax.axis_index("c")
        s = lax.axis_index("s")
        w = c * NS + s
        pltpu.sync_copy(zeros_hbm.at[pl.ds(s * RPT, RPT)],
                        acc.at[pl.ds(s * RPT, RPT)])
        pltpu.sync_copy(g_hbm.at[pl.ds(s * GRT, GRT)],
                        g_sp.at[pl.ds(s * GRT, GRT)])
        pltpu.sync_copy(src_hbm.at[w], src_v)
        pltpu.sync_copy(dst_hbm.at[w], dst_v)
        plsc.subcore_barrier()

        pltpu.async_copy(g_sp.at[src_v.at[0]], rows_a, sga)
        pltpu.async_copy(g_sp.at[src_v.at[1]], rows_b, sgb)

        def body(i, carry):
            j = 2 * i
            pltpu.make_async_copy(g_sp.at[src_v.at[j]], rows_a, sga).wait()
            pltpu.sync_copy(rows_a, acc.at[dst_v.at[j]], add=True)
            pltpu.async_copy(g_sp.at[src_v.at[j + 2]], rows_a, sga)
            pltpu.make_async_copy(g_sp.at[src_v.at[j + 1]], rows_b, sgb).wait()
            pltpu.sync_copy(rows_b, acc.at[dst_v.at[j + 1]], add=True)
            pltpu.async_copy(g_sp.at[src_v.at[j + 3]], rows_b, sgb)
            return carry

        lax.fori_loop(0, (CHUNKS - 3) // 2, body, 0)
        pltpu.make_async_copy(g_sp.at[src_v.at[CHUNKS - 3]], rows_a, sga).wait()
        pltpu.sync_copy(rows_a, acc.at[dst_v.at[CHUNKS - 3]], add=True)
        pltpu.async_copy(g_sp.at[src_v.at[CHUNKS - 1]], rows_a, sga)
        pltpu.make_async_copy(g_sp.at[src_v.at[CHUNKS - 2]], rows_b, sgb).wait()
        pltpu.sync_copy(rows_b, acc.at[dst_v.at[CHUNKS - 2]], add=True)
        pltpu.make_async_copy(g_sp.at[src_v.at[CHUNKS - 1]], rows_a, sga).wait()
        pltpu.sync_copy(rows_a, acc.at[dst_v.at[CHUNKS - 1]], add=True)
        plsc.subcore_barrier()
        pltpu.sync_copy(acc.at[pl.ds(s * RPT, RPT)],
                        out_hbm.at[c, pl.ds(s * RPT, RPT)])

    return lap_kernel


def _dinv_from(deg_ref):
    deg = deg_ref[0] + deg_ref[1]
    return lax.rsqrt(jnp.maximum(deg[:, 0:1], 1.0))


def _bdot(a, b):
    return jnp.dot(a.astype(jnp.bfloat16), b.astype(jnp.bfloat16),
                   preferred_element_type=jnp.float32)


def _tc_pre_body(x_ref, w1_ref, b1_ref, w2_ref, b2_ref, deg_ref, h_ref, g_ref):
    h1 = jnp.maximum(_bdot(x_ref[...], w1_ref[...]) + b1_ref[...], 0.0)
    h2 = jnp.maximum(_bdot(h1, w2_ref[...]) + b2_ref[...], 0.0)
    h_ref[...] = h2
    g_ref[...] = h2 * _dinv_from(deg_ref)


def _tc_pre(x, W1, b1, W2, b2, deg_parts):
    grid = N_NODES // BR
    return pl.pallas_call(
        _tc_pre_body,
        grid=(grid,),
        in_specs=[
            pl.BlockSpec((BR, IN_F), lambda i: (i, 0)),
            pl.BlockSpec((IN_F, H_F), lambda i: (0, 0)),
            pl.BlockSpec((1, H_F), lambda i: (0, 0)),
            pl.BlockSpec((H_F, H_F), lambda i: (0, 0)),
            pl.BlockSpec((1, H_F), lambda i: (0, 0)),
            pl.BlockSpec((NC, BR, 8), lambda i: (0, i, 0)),
        ],
        out_specs=[
            pl.BlockSpec((BR, H_F), lambda i: (i, 0)),
            pl.BlockSpec((BR, H_F), lambda i: (i, 0)),
        ],
        out_shape=[
            jax.ShapeDtypeStruct((N_NODES, H_F), jnp.float32),
            jax.ShapeDtypeStruct((N_NODES, H_F), jnp.float32),
        ],
    )(x, W1, b1, W2, b2, deg_parts)


def _tc_mid_body(h_ref, agg_ref, deg_ref, f_ref, g2_ref):
    dinv = _dinv_from(deg_ref)
    f = h_ref[...] - (agg_ref[0] + agg_ref[1]) * dinv
    f_ref[...] = f
    g2_ref[...] = f * dinv


def _tc_mid(h, agg_parts, deg_parts):
    grid = N_NODES // BR
    return pl.pallas_call(
        _tc_mid_body,
        grid=(grid,),
        in_specs=[
            pl.BlockSpec((BR, H_F), lambda i: (i, 0)),
            pl.BlockSpec((NC, BR, H_F), lambda i: (0, i, 0)),
            pl.BlockSpec((NC, BR, 8), lambda i: (0, i, 0)),
        ],
        out_specs=[
            pl.BlockSpec((BR, H_F), lambda i: (i, 0)),
            pl.BlockSpec((BR, H_F), lambda i: (i, 0)),
        ],
        out_shape=[
            jax.ShapeDtypeStruct((N_NODES, H_F), jnp.float32),
            jax.ShapeDtypeStruct((N_NODES, H_F), jnp.float32),
        ],
    )(h, agg_parts, deg_parts)


def _tc_post_body(h_ref, f1_ref, agg_ref, deg_ref, wm1_ref, bm1_ref,
                  wm2_ref, bm2_ref, out_ref):
    dinv = _dinv_from(deg_ref)
    h = h_ref[...]
    f1 = f1_ref[...]
    f2 = f1 - (agg_ref[0] + agg_ref[1]) * dinv
    out0 = 3.0 * h + (-3.0) * f1 + 0.75 * f2
    out1 = 3.0 * f1 + (-1.5) * f2
    out2 = 0.75 * f2
    hf = jnp.concatenate([out0, out1, out2], axis=1)
    hm = jnp.maximum(_bdot(hf, wm1_ref[...]) + bm1_ref[...], 0.0)
    out_ref[...] = _bdot(hm, wm2_ref[...]) + bm2_ref[...]


def _tc_post(h, f1, agg_parts, deg_parts, Wm1, bm1, Wm2, bm2):
    grid = N_NODES // BR
    return pl.pallas_call(
        _tc_post_body,
        grid=(grid,),
        in_specs=[
            pl.BlockSpec((BR, H_F), lambda i: (i, 0)),
            pl.BlockSpec((BR, H_F), lambda i: (i, 0)),
            pl.BlockSpec((NC, BR, H_F), lambda i: (0, i, 0)),
            pl.BlockSpec((NC, BR, 8), lambda i: (0, i, 0)),
            pl.BlockSpec((3 * H_F, H_F), lambda i: (0, 0)),
            pl.BlockSpec((1, H_F), lambda i: (0, 0)),
            pl.BlockSpec((H_F, 2), lambda i: (0, 0)),
            pl.BlockSpec((1, 2), lambda i: (0, 0)),
        ],
        out_specs=pl.BlockSpec((BR, 2), lambda i: (i, 0)),
        out_shape=jax.ShapeDtypeStruct((N_NODES, 2), jnp.float32),
    )(h, f1, agg_parts, deg_parts, Wm1, bm1, Wm2, bm2)


def kernel(x, edge_index, W1, b1, W2, b2, Wm1, bm1, Wm2, bm2):
    src = edge_index[0].astype(jnp.int32).reshape(NW, CHUNKS, K)
    dst = edge_index[1].astype(jnp.int32).reshape(NW, CHUNKS, K)
    zeros32 = jnp.zeros((NPAD, H_F), jnp.float32)
    zeros8 = jnp.zeros((NPAD, 8), jnp.float32)
    ones8 = jnp.ones((K, 8), jnp.float32)

    deg_parts = _sc_deg()(dst, ones8, zeros8)
    h, g = _tc_pre(x, W1, b1.reshape(1, H_F), W2, b2.reshape(1, H_F),
                   deg_parts)
    agg1 = _sc_lap()(g, src, dst, zeros32)
    f1, g2 = _tc_mid(h, agg1, deg_parts)
    agg2 = _sc_lap()(g2, src, dst, zeros32)
    return _tc_post(h, f1, agg2, deg_parts, Wm1, bm1.reshape(1, H_F),
                    Wm2, bm2.reshape(1, 2))


# speedup vs baseline: 2.2052x; 1.0385x over previous
import functools

import jax
import jax.numpy as jnp
from jax import lax
from jax.experimental import pallas as pl
from jax.experimental.pallas import tpu as pltpu
from jax.experimental.pallas import tpu_sc as plsc

N_NODES = 10000
N_EDGES = 320000
IN_F = 128
H_F = 32
NC = 2
NS = 16
NW = NC * NS
EPW = N_EDGES // NW
K = 80
CHUNKS = EPW // K
NPAD = 10240
RPT = NPAD // NS
GRT = N_NODES // NS
BR = 1000


@functools.cache
def _sc_mesh():
    return plsc.VectorSubcoreMesh(core_axis_name="c", subcore_axis_name="s")


@functools.cache
def _sc_deg():
    @functools.partial(
        pl.kernel,
        out_type=jax.ShapeDtypeStruct((NC, NPAD, 8), jnp.float32),
        mesh=_sc_mesh(),
        scratch_types=[
            pltpu.VMEM_SHARED((NPAD, 8), jnp.float32),
            pltpu.VMEM((CHUNKS, K), jnp.int32),
            pltpu.VMEM((K, 8), jnp.float32),
        ],
        compiler_params=pltpu.CompilerParams(use_tc_tiling_on_sc=False),
    )
    def deg_kernel(dst_hbm, ones_hbm, zeros_hbm, out_hbm, acc, dst_v, ones_v):
        c = lax.axis_index("c")
        s = lax.axis_index("s")
        w = c * NS + s
        pltpu.sync_copy(zeros_hbm.at[pl.ds(s * RPT, RPT)],
                        acc.at[pl.ds(s * RPT, RPT)])
        pltpu.sync_copy(dst_hbm.at[w], dst_v)
        pltpu.sync_copy(ones_hbm, ones_v)
        plsc.subcore_barrier()

        def body(j, carry):
            pltpu.sync_copy(ones_v, acc.at[dst_v.at[j]], add=True)
            return carry

        lax.fori_loop(0, CHUNKS, body, 0)
        plsc.subcore_barrier()
        pltpu.sync_copy(acc.at[pl.ds(s * RPT, RPT)],
                        out_hbm.at[c, pl.ds(s * RPT, RPT)])

    return deg_kernel


@functools.cache
def _sc_lap():
    @functools.partial(
        pl.kernel,
        out_type=jax.ShapeDtypeStruct((NC, NPAD, H_F), jnp.float32),
        mesh=_sc_mesh(),
        scratch_types=[
            pltpu.VMEM_SHARED((NPAD, H_F), jnp.float32),
            pltpu.VMEM_SHARED((N_NODES, H_F), jnp.float32),
            pltpu.VMEM((CHUNKS, K), jnp.int32),
            pltpu.VMEM((CHUNKS, K), jnp.int32),
            pltpu.VMEM((K, H_F), jnp.float32),
            pltpu.VMEM((K, H_F), jnp.float32),
            pltpu.VMEM((K, H_F), jnp.float32),
            pltpu.VMEM((K, H_F), jnp.float32),
            pltpu.SemaphoreType.DMA,
            pltpu.SemaphoreType.DMA,
            pltpu.SemaphoreType.DMA,
            pltpu.SemaphoreType.DMA,
            pltpu.SemaphoreType.DMA,
            pltpu.SemaphoreType.DMA,
            pltpu.SemaphoreType.DMA,
            pltpu.SemaphoreType.DMA,
        ],
        compiler_params=pltpu.CompilerParams(use_tc_tiling_on_sc=False),
    )
    def lap_kernel(g_hbm, src_hbm, dst_hbm, zeros_hbm, out_hbm,
                   acc, g_sp, src_v, dst_v, r0, r1, r2, r3,
                   sg0, sg1, sg2, sg3, ss0, ss1, ss2, ss3):
        c = lax.axis_index("c")
        s = lax.axis_index("s")
        w = c * NS + s
        pltpu.sync_copy(zeros_hbm.at[pl.ds(s * RPT, RPT)],
                        acc.at[pl.ds(s * RPT, RPT)])
        pltpu.sync_copy(g_hbm.at[pl.ds(s * GRT, GRT)],
                        g_sp.at[pl.ds(s * GRT, GRT)])
        pltpu.sync_copy(src_hbm.at[w], src_v)
        pltpu.sync_copy(dst_hbm.at[w], dst_v)
        plsc.subcore_barrier()

        bufs = [r0, r1, r2, r3]
        sgs = [sg0, sg1, sg2, sg3]
        sss = [ss0, ss1, ss2, ss3]

        def gath(j, b):
            pltpu.async_copy(g_sp.at[src_v.at[j]], bufs[b], sgs[b])

        def gath_wait(j, b):
            pltpu.make_async_copy(g_sp.at[src_v.at[j]], bufs[b], sgs[b]).wait()

        def scat(j, b):
            pltpu.async_copy(bufs[b], acc.at[dst_v.at[j]], sss[b], add=True)

        def scat_wait(j, b):
            pltpu.make_async_copy(bufs[b], acc.at[dst_v.at[j]], sss[b]).wait()

        gath(0, 0)
        gath(1, 1)
        gath_wait(0, 0)
        scat(0, 0)
        gath(2, 2)
        gath_wait(1, 1)
        scat(1, 1)
        gath(3, 3)

        def body(i, carry):
            j0 = 2 + 4 * i
            for t in range(4):
                j = j0 + t
                b = (2 + t) % 4
                bn = (b + 2) % 4
                gath_wait(j, b)
                scat(j, b)
                scat_wait(j - 2, bn)
                gath(j + 2, bn)
            return carry

        lax.fori_loop(0, (CHUNKS - 5) // 4, body, 0)
        gath_wait(122, 2)
        scat(122, 2)
        scat_wait(120, 0)
        gath(124, 0)
        gath_wait(123, 3)
        scat(123, 3)
        scat_wait(121, 1)
        gath_wait(124, 0)
        scat(124, 0)
        scat_wait(122, 2)
        scat_wait(123, 3)
        scat_wait(124, 0)
        plsc.subcore_barrier()
        pltpu.sync_copy(acc.at[pl.ds(s * RPT, RPT)],
                        out_hbm.at[c, pl.ds(s * RPT, RPT)])

    return lap_kernel


def _dinv_from(deg_ref):
    deg = deg_ref[0] + deg_ref[1]
    return lax.rsqrt(jnp.maximum(deg[:, 0:1], 1.0))


def _bdot(a, b):
    return jnp.dot(a.astype(jnp.bfloat16), b.astype(jnp.bfloat16),
                   preferred_element_type=jnp.float32)


def _tc_pre_body(x_ref, w1_ref, b1_ref, w2_ref, b2_ref, deg_ref, h_ref, g_ref):
    h1 = jnp.maximum(_bdot(x_ref[...], w1_ref[...]) + b1_ref[...], 0.0)
    h2 = jnp.maximum(_bdot(h1, w2_ref[...]) + b2_ref[...], 0.0)
    h_ref[...] = h2
    g_ref[...] = h2 * _dinv_from(deg_ref)


def _tc_pre(x, W1, b1, W2, b2, deg_parts):
    grid = N_NODES // BR
    return pl.pallas_call(
        _tc_pre_body,
        grid=(grid,),
        in_specs=[
            pl.BlockSpec((BR, IN_F), lambda i: (i, 0)),
            pl.BlockSpec((IN_F, H_F), lambda i: (0, 0)),
            pl.BlockSpec((1, H_F), lambda i: (0, 0)),
            pl.BlockSpec((H_F, H_F), lambda i: (0, 0)),
            pl.BlockSpec((1, H_F), lambda i: (0, 0)),
            pl.BlockSpec((NC, BR, 8), lambda i: (0, i, 0)),
        ],
        out_specs=[
            pl.BlockSpec((BR, H_F), lambda i: (i, 0)),
            pl.BlockSpec((BR, H_F), lambda i: (i, 0)),
        ],
        out_shape=[
            jax.ShapeDtypeStruct((N_NODES, H_F), jnp.float32),
            jax.ShapeDtypeStruct((N_NODES, H_F), jnp.float32),
        ],
    )(x, W1, b1, W2, b2, deg_parts)


def _tc_mid_body(h_ref, agg_ref, deg_ref, f_ref, g2_ref):
    dinv = _dinv_from(deg_ref)
    f = h_ref[...] - (agg_ref[0] + agg_ref[1]) * dinv
    f_ref[...] = f
    g2_ref[...] = f * dinv


def _tc_mid(h, agg_parts, deg_parts):
    grid = N_NODES // BR
    return pl.pallas_call(
        _tc_mid_body,
        grid=(grid,),
        in_specs=[
            pl.BlockSpec((BR, H_F), lambda i: (i, 0)),
            pl.BlockSpec((NC, BR, H_F), lambda i: (0, i, 0)),
            pl.BlockSpec((NC, BR, 8), lambda i: (0, i, 0)),
        ],
        out_specs=[
            pl.BlockSpec((BR, H_F), lambda i: (i, 0)),
            pl.BlockSpec((BR, H_F), lambda i: (i, 0)),
        ],
        out_shape=[
            jax.ShapeDtypeStruct((N_NODES, H_F), jnp.float32),
            jax.ShapeDtypeStruct((N_NODES, H_F), jnp.float32),
        ],
    )(h, agg_parts, deg_parts)


def _tc_post_body(h_ref, f1_ref, agg_ref, deg_ref, wm1_ref, bm1_ref,
                  wm2_ref, bm2_ref, out_ref):
    dinv = _dinv_from(deg_ref)
    h = h_ref[...]
    f1 = f1_ref[...]
    f2 = f1 - (agg_ref[0] + agg_ref[1]) * dinv
    out0 = 3.0 * h + (-3.0) * f1 + 0.75 * f2
    out1 = 3.0 * f1 + (-1.5) * f2
    out2 = 0.75 * f2
    hf = jnp.concatenate([out0, out1, out2], axis=1)
    hm = jnp.maximum(_bdot(hf, wm1_ref[...]) + bm1_ref[...], 0.0)
    out_ref[...] = _bdot(hm, wm2_ref[...]) + bm2_ref[...]


def _tc_post(h, f1, agg_parts, deg_parts, Wm1, bm1, Wm2, bm2):
    grid = N_NODES // BR
    return pl.pallas_call(
        _tc_post_body,
        grid=(grid,),
        in_specs=[
            pl.BlockSpec((BR, H_F), lambda i: (i, 0)),
            pl.BlockSpec((BR, H_F), lambda i: (i, 0)),
            pl.BlockSpec((NC, BR, H_F), lambda i: (0, i, 0)),
            pl.BlockSpec((NC, BR, 8), lambda i: (0, i, 0)),
            pl.BlockSpec((3 * H_F, H_F), lambda i: (0, 0)),
            pl.BlockSpec((1, H_F), lambda i: (0, 0)),
            pl.BlockSpec((H_F, 2), lambda i: (0, 0)),
            pl.BlockSpec((1, 2), lambda i: (0, 0)),
        ],
        out_specs=pl.BlockSpec((BR, 2), lambda i: (i, 0)),
        out_shape=jax.ShapeDtypeStruct((N_NODES, 2), jnp.float32),
    )(h, f1, agg_parts, deg_parts, Wm1, bm1, Wm2, bm2)


def kernel(x, edge_index, W1, b1, W2, b2, Wm1, bm1, Wm2, bm2):
    src = edge_index[0].astype(jnp.int32).reshape(NW, CHUNKS, K)
    dst = edge_index[1].astype(jnp.int32).reshape(NW, CHUNKS, K)
    zeros32 = jnp.zeros((NPAD, H_F), jnp.float32)
    zeros8 = jnp.zeros((NPAD, 8), jnp.float32)
    ones8 = jnp.ones((K, 8), jnp.float32)

    deg_parts = _sc_deg()(dst, ones8, zeros8)
    h, g = _tc_pre(x, W1, b1.reshape(1, H_F), W2, b2.reshape(1, H_F),
                   deg_parts)
    agg1 = _sc_lap()(g, src, dst, zeros32)
    f1, g2 = _tc_mid(h, agg1, deg_parts)
    agg2 = _sc_lap()(g2, src, dst, zeros32)
    return _tc_post(h, f1, agg2, deg_parts, Wm1, bm1.reshape(1, H_F),
                    Wm2, bm2.reshape(1, 2))


# speedup vs baseline: 2.2223x; 1.0077x over previous
import functools

import jax
import jax.numpy as jnp
from jax import lax
from jax.experimental import pallas as pl
from jax.experimental.pallas import tpu as pltpu
from jax.experimental.pallas import tpu_sc as plsc

N_NODES = 10000
N_EDGES = 320000
IN_F = 128
H_F = 32
NC = 2
NS = 16
NW = NC * NS
EPW = N_EDGES // NW
K = 80
CHUNKS = EPW // K
NPAD = 10240
RPT = NPAD // NS
GRT = N_NODES // NS
BR = 1000


@functools.cache
def _sc_mesh():
    return plsc.VectorSubcoreMesh(core_axis_name="c", subcore_axis_name="s")


@functools.cache
def _sc_deg():
    @functools.partial(
        pl.kernel,
        out_type=jax.ShapeDtypeStruct((NC, NPAD, 8), jnp.float32),
        mesh=_sc_mesh(),
        scratch_types=[
            pltpu.VMEM_SHARED((NPAD, 8), jnp.float32),
            pltpu.VMEM((CHUNKS, K), jnp.int32),
            pltpu.VMEM((K, 8), jnp.float32),
        ],
        compiler_params=pltpu.CompilerParams(use_tc_tiling_on_sc=False),
    )
    def deg_kernel(dst_hbm, ones_hbm, zeros_hbm, out_hbm, acc, dst_v, ones_v):
        c = lax.axis_index("c")
        s = lax.axis_index("s")
        w = c * NS + s
        pltpu.sync_copy(zeros_hbm.at[pl.ds(s * RPT, RPT)],
                        acc.at[pl.ds(s * RPT, RPT)])
        pltpu.sync_copy(dst_hbm.at[w], dst_v)
        pltpu.sync_copy(ones_hbm, ones_v)
        plsc.subcore_barrier()

        def body(j, carry):
            pltpu.sync_copy(ones_v, acc.at[dst_v.at[j]], add=True)
            return carry

        lax.fori_loop(0, CHUNKS, body, 0)
        plsc.subcore_barrier()
        pltpu.sync_copy(acc.at[pl.ds(s * RPT, RPT)],
                        out_hbm.at[c, pl.ds(s * RPT, RPT)])

    return deg_kernel


@functools.cache
def _sc_lap():
    @functools.partial(
        pl.kernel,
        out_type=jax.ShapeDtypeStruct((NC, NPAD, H_F), jnp.float32),
        mesh=_sc_mesh(),
        scratch_types=[
            pltpu.VMEM_SHARED((NPAD, H_F), jnp.float32),
            pltpu.VMEM_SHARED((N_NODES, H_F), jnp.float32),
            pltpu.VMEM((CHUNKS, K), jnp.int32),
            pltpu.VMEM((CHUNKS, K), jnp.int32),
            pltpu.VMEM((K, H_F), jnp.float32),
            pltpu.VMEM((K, H_F), jnp.float32),
            pltpu.VMEM((K, H_F), jnp.float32),
            pltpu.VMEM((K, H_F), jnp.float32),
            pltpu.SemaphoreType.DMA,
            pltpu.SemaphoreType.DMA,
            pltpu.SemaphoreType.DMA,
            pltpu.SemaphoreType.DMA,
            pltpu.SemaphoreType.DMA,
            pltpu.SemaphoreType.DMA,
            pltpu.SemaphoreType.DMA,
            pltpu.SemaphoreType.DMA,
        ],
        compiler_params=pltpu.CompilerParams(use_tc_tiling_on_sc=False),
    )
    def lap_kernel(g_hbm, src_hbm, dst_hbm, zeros_hbm, out_hbm,
                   acc, g_sp, src_v, dst_v, r0, r1, r2, r3,
                   sg0, sg1, sg2, sg3, ss0, ss1, ss2, ss3):
        c = lax.axis_index("c")
        s = lax.axis_index("s")
        w = c * NS + s
        pltpu.sync_copy(zeros_hbm.at[pl.ds(s * RPT, RPT)],
                        acc.at[pl.ds(s * RPT, RPT)])
        pltpu.sync_copy(g_hbm.at[pl.ds(s * GRT, GRT)],
                        g_sp.at[pl.ds(s * GRT, GRT)])
        pltpu.sync_copy(src_hbm.at[w], src_v)
        pltpu.sync_copy(dst_hbm.at[w], dst_v)
        plsc.subcore_barrier()

        bufs = [r0, r1, r2, r3]
        sgs = [sg0, sg1, sg2, sg3]
        sss = [ss0, ss1, ss2, ss3]

        def gath(j, b):
            pltpu.async_copy(g_sp.at[src_v.at[j]], bufs[b], sgs[b])

        def gath_wait(j, b):
            pltpu.make_async_copy(g_sp.at[src_v.at[j]], bufs[b], sgs[b]).wait()

        def scat(j, b):
            pltpu.async_copy(bufs[b], acc.at[dst_v.at[j]], sss[b], add=True)

        def scat_wait(j, b):
            pltpu.make_async_copy(bufs[b], acc.at[dst_v.at[j]], sss[b]).wait()

        gath(0, 0)
        gath(1, 1)
        gath_wait(0, 0)
        scat(0, 0)
        gath(2, 2)
        gath_wait(1, 1)
        scat(1, 1)
        gath(3, 3)

        def body(i, carry):
            j0 = 2 + 4 * i
            for t in range(4):
                j = j0 + t
                b = (2 + t) % 4
                bn = (b + 2) % 4
                gath_wait(j, b)
                scat(j, b)
                scat_wait(j - 2, bn)
                gath(j + 2, bn)
            return carry

        lax.fori_loop(0, (CHUNKS - 5) // 4, body, 0)
        gath_wait(122, 2)
        scat(122, 2)
        scat_wait(120, 0)
        gath(124, 0)
        gath_wait(123, 3)
        scat(123, 3)
        scat_wait(121, 1)
        gath_wait(124, 0)
        scat(124, 0)
        scat_wait(122, 2)
        scat_wait(123, 3)
        scat_wait(124, 0)
        plsc.subcore_barrier()
        pltpu.sync_copy(acc.at[pl.ds(s * RPT, RPT)],
                        out_hbm.at[c, pl.ds(s * RPT, RPT)])

    return lap_kernel


def _dinv_from(deg_ref):
    deg = deg_ref[0] + deg_ref[1]
    return lax.rsqrt(jnp.maximum(deg[:, 0:1], 1.0))


def _bdot(a, b):
    return jnp.dot(a.astype(jnp.bfloat16), b.astype(jnp.bfloat16),
                   preferred_element_type=jnp.float32)


def _tc_mm_body(x_ref, w1_ref, b1_ref, w2_ref, b2_ref, h_ref):
    h1 = jnp.maximum(_bdot(x_ref[...], w1_ref[...]) + b1_ref[...], 0.0)
    h_ref[...] = jnp.maximum(_bdot(h1, w2_ref[...]) + b2_ref[...], 0.0)


def _tc_mm(x, W1, b1, W2, b2):
    grid = N_NODES // BR
    return pl.pallas_call(
        _tc_mm_body,
        grid=(grid,),
        in_specs=[
            pl.BlockSpec((BR, IN_F), lambda i: (i, 0)),
            pl.BlockSpec((IN_F, H_F), lambda i: (0, 0)),
            pl.BlockSpec((1, H_F), lambda i: (0, 0)),
            pl.BlockSpec((H_F, H_F), lambda i: (0, 0)),
            pl.BlockSpec((1, H_F), lambda i: (0, 0)),
        ],
        out_specs=pl.BlockSpec((BR, H_F), lambda i: (i, 0)),
        out_shape=jax.ShapeDtypeStruct((N_NODES, H_F), jnp.float32),
    )(x, W1, b1, W2, b2)


def _tc_norm_body(h_ref, deg_ref, g_ref):
    g_ref[...] = h_ref[...] * _dinv_from(deg_ref)


def _tc_norm(h, deg_parts):
    grid = N_NODES // BR
    return pl.pallas_call(
        _tc_norm_body,
        grid=(grid,),
        in_specs=[
            pl.BlockSpec((BR, H_F), lambda i: (i, 0)),
            pl.BlockSpec((NC, BR, 8), lambda i: (0, i, 0)),
        ],
        out_specs=pl.BlockSpec((BR, H_F), lambda i: (i, 0)),
        out_shape=jax.ShapeDtypeStruct((N_NODES, H_F), jnp.float32),
    )(h, deg_parts)


def _tc_mid_body(h_ref, agg_ref, deg_ref, f_ref, g2_ref):
    dinv = _dinv_from(deg_ref)
    f = h_ref[...] - (agg_ref[0] + agg_ref[1]) * dinv
    f_ref[...] = f
    g2_ref[...] = f * dinv


def _tc_mid(h, agg_parts, deg_parts):
    grid = N_NODES // BR
    return pl.pallas_call(
        _tc_mid_body,
        grid=(grid,),
        in_specs=[
            pl.BlockSpec((BR, H_F), lambda i: (i, 0)),
            pl.BlockSpec((NC, BR, H_F), lambda i: (0, i, 0)),
            pl.BlockSpec((NC, BR, 8), lambda i: (0, i, 0)),
        ],
        out_specs=[
            pl.BlockSpec((BR, H_F), lambda i: (i, 0)),
            pl.BlockSpec((BR, H_F), lambda i: (i, 0)),
        ],
        out_shape=[
            jax.ShapeDtypeStruct((N_NODES, H_F), jnp.float32),
            jax.ShapeDtypeStruct((N_NODES, H_F), jnp.float32),
        ],
    )(h, agg_parts, deg_parts)


def _tc_post_body(h_ref, f1_ref, agg_ref, deg_ref, wm1_ref, bm1_ref,
                  wm2_ref, bm2_ref, out_ref):
    dinv = _dinv_from(deg_ref)
    h = h_ref[...]
    f1 = f1_ref[...]
    f2 = f1 - (agg_ref[0] + agg_ref[1]) * dinv
    out0 = 3.0 * h + (-3.0) * f1 + 0.75 * f2
    out1 = 3.0 * f1 + (-1.5) * f2
    out2 = 0.75 * f2
    hf = jnp.concatenate([out0, out1, out2], axis=1)
    hm = jnp.maximum(_bdot(hf, wm1_ref[...]) + bm1_ref[...], 0.0)
    out_ref[...] = _bdot(hm, wm2_ref[...]) + bm2_ref[...]


def _tc_post(h, f1, agg_parts, deg_parts, Wm1, bm1, Wm2, bm2):
    grid = N_NODES // BR
    return pl.pallas_call(
        _tc_post_body,
        grid=(grid,),
        in_specs=[
            pl.BlockSpec((BR, H_F), lambda i: (i, 0)),
            pl.BlockSpec((BR, H_F), lambda i: (i, 0)),
            pl.BlockSpec((NC, BR, H_F), lambda i: (0, i, 0)),
            pl.BlockSpec((NC, BR, 8), lambda i: (0, i, 0)),
            pl.BlockSpec((3 * H_F, H_F), lambda i: (0, 0)),
            pl.BlockSpec((1, H_F), lambda i: (0, 0)),
            pl.BlockSpec((H_F, 2), lambda i: (0, 0)),
            pl.BlockSpec((1, 2), lambda i: (0, 0)),
        ],
        out_specs=pl.BlockSpec((BR, 2), lambda i: (i, 0)),
        out_shape=jax.ShapeDtypeStruct((N_NODES, 2), jnp.float32),
    )(h, f1, agg_parts, deg_parts, Wm1, bm1, Wm2, bm2)


def kernel(x, edge_index, W1, b1, W2, b2, Wm1, bm1, Wm2, bm2):
    src = edge_index[0].astype(jnp.int32).reshape(NW, CHUNKS, K)
    dst = edge_index[1].astype(jnp.int32).reshape(NW, CHUNKS, K)
    zeros32 = jnp.zeros((NPAD, H_F), jnp.float32)
    zeros8 = jnp.zeros((NPAD, 8), jnp.float32)
    ones8 = jnp.ones((K, 8), jnp.float32)

    deg_parts = _sc_deg()(dst, ones8, zeros8)
    h = _tc_mm(x, W1, b1.reshape(1, H_F), W2, b2.reshape(1, H_F))
    g = _tc_norm(h, deg_parts)
    agg1 = _sc_lap()(g, src, dst, zeros32)
    f1, g2 = _tc_mid(h, agg1, deg_parts)
    agg2 = _sc_lap()(g2, src, dst, zeros32)
    return _tc_post(h, f1, agg2, deg_parts, Wm1, bm1.reshape(1, H_F),
                    Wm2, bm2.reshape(1, 2))
